# baseline (speedup 1.0000x reference)
import jax
import jax.numpy as jnp
from jax import lax
from jax.experimental import pallas as pl
from jax.experimental.pallas import tpu as pltpu
from jax.experimental.pallas import tpu_sc as plsc

CLS = 1000
FEAT = 64
BATCH = 1024
CLS_P = 1024

_L = 16
_NW = 32
_CPW = CLS_P // _NW
_BPW = BATCH // _NW
_NCHUNK = BATCH // _L


def _shift_cmp(idx, iota, s):
    gidx = jnp.minimum(iota + s, _L - 1)
    shifted = lax.gather(
        idx, gidx[:, None],
        lax.GatherDimensionNumbers(
            offset_dims=(), collapsed_slice_dims=(0,), start_index_map=(0,)),
        slice_sizes=(1,),
        mode=lax.GatherScatterMode.PROMISE_IN_BOUNDS)
    return (shifted == idx) & (iota < _L - s)


def _sc_body(table_hbm, cls_hbm, sel_hbm, tmp_hbm,
             cls_v, lp_v, gidx_v, sidx_v, rows_v, trows_v, sem, sem2):
    wid = lax.axis_index("c") * 16 + lax.axis_index("s")
    base = wid * _BPW
    cbase = wid * _CPW
    iota = lax.iota(jnp.int32, _L)

    pltpu.sync_copy(cls_hbm, cls_v)

    gidx_v[pl.ds(0, _L)] = cls_v[pl.ds(base, _L)] + BATCH
    gidx_v[pl.ds(_L, _L)] = cls_v[pl.ds(base + _L, _L)] + BATCH
    sel_dma = pltpu.async_copy(table_hbm.at[gidx_v], rows_v, sem)

    def _init(j, carry):
        lp_v[pl.ds(j * _L, _L)] = jnp.full((_L,), -1, jnp.int32)
        return carry
    lax.fori_loop(0, CLS_P // _L, _init, 0)

    def _scatter_chunk(i, carry):
        idx = cls_v[pl.ds(i * _L, _L)]
        pos = iota + i * _L
        dup = iota < 0
        for s in range(1, _L):
            dup = dup | _shift_cmp(idx, iota, s)
        plsc.store_scatter(lp_v, [idx], pos, mask=jnp.logical_not(dup))
        return carry
    lax.fori_loop(0, _NCHUNK, _scatter_chunk, 0)

    lp0 = lp_v[pl.ds(cbase, _L)]
    lp1 = lp_v[pl.ds(cbase + _L, _L)]
    sidx_v[pl.ds(0, _L)] = jnp.where(
        lp0 >= 0, lp0, cbase + BATCH + iota)
    sidx_v[pl.ds(_L, _L)] = jnp.where(
        lp1 >= 0, lp1, cbase + BATCH + _L + iota)

    pltpu.async_copy(table_hbm.at[sidx_v], trows_v, sem2).wait()
    pltpu.sync_copy(trows_v, tmp_hbm.at[pl.ds(cbase, _CPW)])

    sel_dma.wait()
    pltpu.sync_copy(rows_v, sel_hbm.at[pl.ds(base, _BPW)])


def _bcast_body(tmp_ref, out_ref):
    out_ref[...] = jnp.broadcast_to(tmp_ref[...][None], out_ref.shape)


_BB = 16


def kernel(x, cls_list, memory):
    table = jnp.concatenate(
        [x, memory, jnp.zeros((CLS_P - CLS, FEAT), memory.dtype)], axis=0)

    mesh = plsc.VectorSubcoreMesh(core_axis_name="c", subcore_axis_name="s")
    sc = pl.kernel(
        _sc_body,
        mesh=mesh,
        out_type=(
            jax.ShapeDtypeStruct((BATCH, FEAT), jnp.float32),
            jax.ShapeDtypeStruct((CLS_P, FEAT), jnp.float32),
        ),
        scratch_types=[
            pltpu.VMEM((BATCH,), jnp.int32),
            pltpu.VMEM((CLS_P,), jnp.int32),
            pltpu.VMEM((_BPW,), jnp.int32),
            pltpu.VMEM((_CPW,), jnp.int32),
            pltpu.VMEM((_BPW, FEAT), jnp.float32),
            pltpu.VMEM((_CPW, FEAT), jnp.float32),
            pltpu.SemaphoreType.DMA,
            pltpu.SemaphoreType.DMA,
        ],
        compiler_params=pltpu.CompilerParams(
            needs_layout_passes=False, use_tc_tiling_on_sc=False),
    )
    selected, tmp_pad = sc(table, cls_list)
    tmp = tmp_pad[:CLS]

    tmp_rep = pl.pallas_call(
        _bcast_body,
        grid=(BATCH // _BB,),
        in_specs=[pl.BlockSpec((CLS, FEAT), lambda i: (0, 0))],
        out_specs=pl.BlockSpec((_BB, CLS, FEAT), lambda i: (i, 0, 0)),
        out_shape=jax.ShapeDtypeStruct((BATCH, CLS, FEAT), jnp.float32),
    )(tmp)

    return selected, tmp_rep


# speedup vs baseline: 1.0015x; 1.0015x over previous
import jax
import jax.numpy as jnp
from jax import lax
from jax.experimental import pallas as pl
from jax.experimental.pallas import tpu as pltpu
from jax.experimental.pallas import tpu_sc as plsc

CLS = 1000
FEAT = 64
BATCH = 1024
CLS_P = 1024

_L = 16
_NW = 32
_CPW = CLS_P // _NW
_BPW = BATCH // _NW
_NCHUNK = BATCH // _L


def _shift_cmp(idx, iota, s):
    gidx = jnp.minimum(iota + s, _L - 1)
    shifted = lax.gather(
        idx, gidx[:, None],
        lax.GatherDimensionNumbers(
            offset_dims=(), collapsed_slice_dims=(0,), start_index_map=(0,)),
        slice_sizes=(1,),
        mode=lax.GatherScatterMode.PROMISE_IN_BOUNDS)
    return (shifted == idx) & (iota < _L - s)


def _sc_body(table_hbm, cls_hbm, sel_hbm, tmp_hbm,
             cls_v, lp_v, gidx_v, sidx_v, rows_v, trows_v, sem, sem2):
    wid = lax.axis_index("c") * 16 + lax.axis_index("s")
    base = wid * _BPW
    cbase = wid * _CPW
    iota = lax.iota(jnp.int32, _L)

    pltpu.sync_copy(cls_hbm, cls_v)

    gidx_v[pl.ds(0, _L)] = cls_v[pl.ds(base, _L)] + BATCH
    gidx_v[pl.ds(_L, _L)] = cls_v[pl.ds(base + _L, _L)] + BATCH
    sel_dma = pltpu.async_copy(table_hbm.at[gidx_v], rows_v, sem)

    def _init(j, carry):
        lp_v[pl.ds(j * _L, _L)] = jnp.full((_L,), -1, jnp.int32)
        return carry
    lax.fori_loop(0, CLS_P // _L, _init, 0)

    def _scatter_chunk(i, carry):
        idx = cls_v[pl.ds(i * _L, _L)]
        pos = iota + i * _L
        dup = iota < 0
        for s in range(1, _L):
            dup = dup | _shift_cmp(idx, iota, s)
        plsc.store_scatter(lp_v, [idx], pos, mask=jnp.logical_not(dup))
        return carry
    lax.fori_loop(0, _NCHUNK, _scatter_chunk, 0)

    lp0 = lp_v[pl.ds(cbase, _L)]
    lp1 = lp_v[pl.ds(cbase + _L, _L)]
    sidx_v[pl.ds(0, _L)] = jnp.where(
        lp0 >= 0, lp0, cbase + BATCH + iota)
    sidx_v[pl.ds(_L, _L)] = jnp.where(
        lp1 >= 0, lp1, cbase + BATCH + _L + iota)

    pltpu.async_copy(table_hbm.at[sidx_v], trows_v, sem2).wait()

    @pl.when(wid < _NW - 1)
    def _full_block():
        pltpu.sync_copy(trows_v, tmp_hbm.at[pl.ds(cbase, _CPW)])

    @pl.when(wid == _NW - 1)
    def _tail_block():
        pltpu.sync_copy(trows_v.at[pl.ds(0, CLS - (_NW - 1) * _CPW)],
                        tmp_hbm.at[pl.ds(cbase, CLS - (_NW - 1) * _CPW)])

    sel_dma.wait()
    pltpu.sync_copy(rows_v, sel_hbm.at[pl.ds(base, _BPW)])


_BB = 16
_NSEM = 8


def _bcast_body(tmp_ref, out_ref, rep_ref, sems):
    for j in range(_BB):
        rep_ref[j] = tmp_ref[...]
    ncopies = BATCH // _BB
    for i in range(ncopies):
        if i >= _NSEM:
            pltpu.make_async_copy(
                rep_ref, out_ref.at[pl.ds((i - _NSEM) * _BB, _BB)],
                sems.at[i % _NSEM]).wait()
        pltpu.make_async_copy(
            rep_ref, out_ref.at[pl.ds(i * _BB, _BB)],
            sems.at[i % _NSEM]).start()
    for i in range(ncopies - _NSEM, ncopies):
        pltpu.make_async_copy(
            rep_ref, out_ref.at[pl.ds(i * _BB, _BB)],
            sems.at[i % _NSEM]).wait()


def kernel(x, cls_list, memory):
    table = jnp.concatenate(
        [x, memory, jnp.zeros((CLS_P - CLS, FEAT), memory.dtype)], axis=0)

    mesh = plsc.VectorSubcoreMesh(core_axis_name="c", subcore_axis_name="s")
    sc = pl.kernel(
        _sc_body,
        mesh=mesh,
        out_type=(
            jax.ShapeDtypeStruct((BATCH, FEAT), jnp.float32),
            jax.ShapeDtypeStruct((CLS, FEAT), jnp.float32),
        ),
        scratch_types=[
            pltpu.VMEM((BATCH,), jnp.int32),
            pltpu.VMEM((CLS_P,), jnp.int32),
            pltpu.VMEM((_BPW,), jnp.int32),
            pltpu.VMEM((_CPW,), jnp.int32),
            pltpu.VMEM((_BPW, FEAT), jnp.float32),
            pltpu.VMEM((_CPW, FEAT), jnp.float32),
            pltpu.SemaphoreType.DMA,
            pltpu.SemaphoreType.DMA,
        ],
        compiler_params=pltpu.CompilerParams(
            needs_layout_passes=False, use_tc_tiling_on_sc=False),
    )
    selected, tmp = sc(table, cls_list)

    tmp_rep = pl.pallas_call(
        _bcast_body,
        in_specs=[pl.BlockSpec(memory_space=pltpu.VMEM)],
        out_specs=pl.BlockSpec(memory_space=pl.ANY),
        out_shape=jax.ShapeDtypeStruct((BATCH, CLS, FEAT), jnp.float32),
        scratch_shapes=[
            pltpu.VMEM((_BB, CLS, FEAT), jnp.float32),
            pltpu.SemaphoreType.DMA((_NSEM,)),
        ],
    )(tmp)

    return selected, tmp_rep


# speedup vs baseline: 1.0083x; 1.0067x over previous
import jax
import jax.numpy as jnp
from jax import lax
from jax.experimental import pallas as pl
from jax.experimental.pallas import tpu as pltpu
from jax.experimental.pallas import tpu_sc as plsc

CLS = 1000
FEAT = 64
BATCH = 1024
CLS_P = 1024

_L = 16
_NW = 32
_CPW = CLS_P // _NW
_BPW = BATCH // _NW
_NCHUNK = BATCH // _L


def _shift_cmp(idx, iota, s):
    gidx = jnp.minimum(iota + s, _L - 1)
    shifted = lax.gather(
        idx, gidx[:, None],
        lax.GatherDimensionNumbers(
            offset_dims=(), collapsed_slice_dims=(0,), start_index_map=(0,)),
        slice_sizes=(1,),
        mode=lax.GatherScatterMode.PROMISE_IN_BOUNDS)
    return (shifted == idx) & (iota < _L - s)


def _sc_body(table_hbm, cls_hbm, sel_hbm, tmp_hbm,
             cls_v, lp_v, gidx_v, sidx_v, rows_v, trows_v, sem, sem2):
    wid = lax.axis_index("c") * 16 + lax.axis_index("s")
    base = wid * _BPW
    cbase = wid * _CPW
    iota = lax.iota(jnp.int32, _L)

    pltpu.sync_copy(cls_hbm, cls_v)

    gidx_v[pl.ds(0, _L)] = cls_v[pl.ds(base, _L)] + BATCH
    gidx_v[pl.ds(_L, _L)] = cls_v[pl.ds(base + _L, _L)] + BATCH
    sel_dma = pltpu.async_copy(table_hbm.at[gidx_v], rows_v, sem)

    def _init(j, carry):
        lp_v[pl.ds(j * _L, _L)] = jnp.full((_L,), -1, jnp.int32)
        return carry
    lax.fori_loop(0, CLS_P // _L, _init, 0)

    def _scatter_chunk(i, carry):
        idx = cls_v[pl.ds(i * _L, _L)]
        pos = iota + i * _L
        dup = iota < 0
        for s in range(1, _L):
            dup = dup | _shift_cmp(idx, iota, s)
        plsc.store_scatter(lp_v, [idx], pos, mask=jnp.logical_not(dup))
        return carry
    lax.fori_loop(0, _NCHUNK, _scatter_chunk, 0)

    lp0 = lp_v[pl.ds(cbase, _L)]
    lp1 = lp_v[pl.ds(cbase + _L, _L)]
    sidx_v[pl.ds(0, _L)] = jnp.where(
        lp0 >= 0, lp0, cbase + BATCH + iota)
    sidx_v[pl.ds(_L, _L)] = jnp.where(
        lp1 >= 0, lp1, cbase + BATCH + _L + iota)

    pltpu.async_copy(table_hbm.at[sidx_v], trows_v, sem2).wait()

    @pl.when(wid < _NW - 1)
    def _full_block():
        pltpu.sync_copy(trows_v, tmp_hbm.at[pl.ds(cbase, _CPW)])

    @pl.when(wid == _NW - 1)
    def _tail_block():
        pltpu.sync_copy(trows_v.at[pl.ds(0, CLS - (_NW - 1) * _CPW)],
                        tmp_hbm.at[pl.ds(cbase, CLS - (_NW - 1) * _CPW)])

    sel_dma.wait()
    pltpu.sync_copy(rows_v, sel_hbm.at[pl.ds(base, _BPW)])


_BB = 16


def _bcast_body(tmp_ref, out_ref):
    out_ref[...] = jnp.broadcast_to(tmp_ref[...][None], (_BB, CLS, FEAT))


def kernel(x, cls_list, memory):
    table = jnp.concatenate(
        [x, memory, jnp.zeros((CLS_P - CLS, FEAT), memory.dtype)], axis=0)

    mesh = plsc.VectorSubcoreMesh(core_axis_name="c", subcore_axis_name="s")
    sc = pl.kernel(
        _sc_body,
        mesh=mesh,
        out_type=(
            jax.ShapeDtypeStruct((BATCH, FEAT), jnp.float32),
            jax.ShapeDtypeStruct((CLS, FEAT), jnp.float32),
        ),
        scratch_types=[
            pltpu.VMEM((BATCH,), jnp.int32),
            pltpu.VMEM((CLS_P,), jnp.int32),
            pltpu.VMEM((_BPW,), jnp.int32),
            pltpu.VMEM((_CPW,), jnp.int32),
            pltpu.VMEM((_BPW, FEAT), jnp.float32),
            pltpu.VMEM((_CPW, FEAT), jnp.float32),
            pltpu.SemaphoreType.DMA,
            pltpu.SemaphoreType.DMA,
        ],
        compiler_params=pltpu.CompilerParams(
            needs_layout_passes=False, use_tc_tiling_on_sc=False),
    )
    selected, tmp = sc(table, cls_list)

    tmp_rep = pl.pallas_call(
        _bcast_body,
        grid=(BATCH // _BB,),
        in_specs=[pl.BlockSpec((CLS, FEAT), lambda i: (0, 0))],
        out_specs=pl.BlockSpec((_BB, CLS, FEAT), lambda i: (i, 0, 0)),
        out_shape=jax.ShapeDtypeStruct((BATCH, CLS, FEAT), jnp.float32),
        compiler_params=pltpu.CompilerParams(
            dimension_semantics=("arbitrary",)),
    )(tmp)

    return selected, tmp_rep
